# Initial kernel scaffold; baseline (speedup 1.0000x reference)
import jax
import jax.numpy as jnp
from jax.experimental import pallas as pl


def kernel(x, Wg, bg, W1, b1, W2, b2):
    raise NotImplementedError("write your pallas kernel here")


# speedup vs baseline: 7.2531x; 7.2531x over previous
import functools

import jax
import jax.numpy as jnp
from jax import lax
from jax.experimental import pallas as pl
from jax.experimental.pallas import tpu as pltpu
from jax.experimental.pallas import tpu_sc as plsc

E = 64
H = 768
F = 3072
N = 8192
T = 128
NT = N // T + E
NP = NT * T
TB = 1024
G = N // TB


def _gate_body(x_ref, wg_ref, bg_ref, eid_ref, psum_ref, cnt_ref):
    xb = x_ref[...]
    logits = jnp.dot(xb, wg_ref[...], preferred_element_type=jnp.float32)
    logits = logits + bg_ref[...]
    m = jnp.max(logits, axis=-1, keepdims=True)
    ex = jnp.exp(logits - m)
    s = jnp.sum(ex, axis=-1, keepdims=True)
    probs = ex / s
    ids = lax.broadcasted_iota(jnp.int32, logits.shape, 1)
    eid = jnp.min(jnp.where(logits == m, ids, E), axis=-1, keepdims=True)
    eid_ref[...] = eid
    psum_ref[...] = jnp.sum(probs, axis=0, keepdims=True)[None]
    onehot = (ids == eid).astype(jnp.float32)
    cnt_ref[...] = jnp.sum(onehot, axis=0, keepdims=True)[None]


def _gate_call(x, Wg, bg2):
    return pl.pallas_call(
        _gate_body,
        grid=(G,),
        in_specs=[
            pl.BlockSpec((TB, H), lambda i: (i, 0)),
            pl.BlockSpec((H, E), lambda i: (0, 0)),
            pl.BlockSpec((1, E), lambda i: (0, 0)),
        ],
        out_specs=[
            pl.BlockSpec((TB, 1), lambda i: (i, 0)),
            pl.BlockSpec((1, 1, E), lambda i: (i, 0, 0)),
            pl.BlockSpec((1, 1, E), lambda i: (i, 0, 0)),
        ],
        out_shape=[
            jax.ShapeDtypeStruct((N, 1), jnp.int32),
            jax.ShapeDtypeStruct((G, 1, E), jnp.float32),
            jax.ShapeDtypeStruct((G, 1, E), jnp.float32),
        ],
    )(x, Wg, bg2)


def _route_metadata(eid, counts):
    perm = jnp.argsort(eid).astype(jnp.int32)
    ntiles = (counts + (T - 1)) // T
    cum_tiles = jnp.cumsum(ntiles)
    padded_start = (cum_tiles - ntiles) * T
    orig_start = jnp.cumsum(counts) - counts
    r = jnp.arange(N, dtype=jnp.int32)
    e_sorted = eid[perm]
    slot = padded_start[e_sorted] + (r - orig_start[e_sorted])
    src_idx = jnp.zeros((NP,), jnp.int32).at[slot].set(perm)
    pos = jnp.zeros((N,), jnp.int32).at[perm].set(slot)
    tile_ids = jnp.arange(NT, dtype=jnp.int32)
    expert_tile = jnp.minimum(
        jnp.searchsorted(cum_tiles, tile_ids, side="right"), E - 1
    ).astype(jnp.int32)
    return src_idx, pos, expert_tile


def _expert_body(et_ref, xs_ref, w1_ref, b1_ref, w2_ref, b2_ref,
                 wg_ref, bg_ref, ys_ref):
    del et_ref
    xb = xs_ref[...]
    logits = jnp.dot(xb, wg_ref[...], preferred_element_type=jnp.float32)
    logits = logits + bg_ref[...]
    m = jnp.max(logits, axis=-1, keepdims=True)
    p = 1.0 / jnp.sum(jnp.exp(logits - m), axis=-1, keepdims=True)
    h = jnp.dot(xb, w1_ref[0], preferred_element_type=jnp.float32)
    h = jnp.maximum(h + b1_ref[0], 0.0)
    o = jnp.dot(h, w2_ref[0], preferred_element_type=jnp.float32)
    ys_ref[...] = (o + b2_ref[0]) * p


def _expert_call(expert_tile, xs, W1, b1, W2, b2, Wg, bg2):
    grid_spec = pltpu.PrefetchScalarGridSpec(
        num_scalar_prefetch=1,
        grid=(NT,),
        in_specs=[
            pl.BlockSpec((T, H), lambda i, et: (i, 0)),
            pl.BlockSpec((1, H, F), lambda i, et: (et[i], 0, 0)),
            pl.BlockSpec((1, 1, F), lambda i, et: (et[i], 0, 0)),
            pl.BlockSpec((1, F, H), lambda i, et: (et[i], 0, 0)),
            pl.BlockSpec((1, 1, H), lambda i, et: (et[i], 0, 0)),
            pl.BlockSpec((H, E), lambda i, et: (0, 0)),
            pl.BlockSpec((1, E), lambda i, et: (0, 0)),
        ],
        out_specs=pl.BlockSpec((T, H), lambda i, et: (i, 0)),
    )
    return pl.pallas_call(
        _expert_body,
        grid_spec=grid_spec,
        out_shape=jax.ShapeDtypeStruct((NP, H), jnp.float32),
        compiler_params=pltpu.CompilerParams(
            dimension_semantics=("arbitrary",),
        ),
    )(expert_tile, xs, W1, b1, W2, b2, Wg, bg2)


@functools.lru_cache(maxsize=None)
def _make_sc_gather(n_rows_out, d):
    info = plsc.get_sparse_core_info()
    _NC = info.num_cores
    _NW = info.num_cores * info.num_subcores
    b_per_w = n_rows_out // _NW
    ch = 64 if b_per_w % 64 == 0 else b_per_w
    n_ch = b_per_w // ch
    mesh = plsc.VectorSubcoreMesh(core_axis_name="c", subcore_axis_name="s")

    @functools.partial(
        pl.kernel,
        mesh=mesh,
        out_type=jax.ShapeDtypeStruct((n_rows_out, d), jnp.float32),
        scratch_types=[
            pltpu.VMEM((b_per_w,), jnp.int32),
            pltpu.VMEM((ch, d), jnp.float32),
            pltpu.SemaphoreType.DMA,
        ],
    )
    def k(table_hbm, idx_hbm, out_hbm, idx_v, rows_v, sem):
        wid = lax.axis_index("s") * _NC + lax.axis_index("c")
        base = wid * b_per_w
        pltpu.sync_copy(idx_hbm.at[pl.ds(base, b_per_w)], idx_v)
        for c in range(n_ch):
            pltpu.async_copy(
                table_hbm.at[idx_v.at[pl.ds(c * ch, ch)]], rows_v, sem
            ).wait()
            pltpu.sync_copy(rows_v, out_hbm.at[pl.ds(base + c * ch, ch)])

    return k


def _gather_rows(table, idx):
    return _make_sc_gather(idx.shape[0], table.shape[1])(table, idx)


def kernel(x, Wg, bg, W1, b1, W2, b2):
    bg2 = bg.reshape(1, E)
    eid2, psum, cnt = _gate_call(x, Wg, bg2)
    eid = eid2[:, 0]
    probs_sum = psum.sum(axis=(0, 1))
    counts_f = cnt.sum(axis=(0, 1))
    aux = jnp.dot(probs_sum / N, counts_f / N) * E
    counts = counts_f.astype(jnp.int32)
    src_idx, pos, expert_tile = _route_metadata(eid, counts)
    xs = _gather_rows(x, src_idx)
    ys = _expert_call(expert_tile, xs, W1, b1[:, None, :], W2,
                      b2[:, None, :], Wg, bg2)
    y = _gather_rows(ys, pos)
    return y, aux


# speedup vs baseline: 10.5712x; 1.4575x over previous
import functools

import jax
import jax.numpy as jnp
from jax import lax
from jax.experimental import pallas as pl
from jax.experimental.pallas import tpu as pltpu
from jax.experimental.pallas import tpu_sc as plsc

E = 64
H = 768
F = 3072
N = 8192
T = 128
NT = N // T + E
NP = NT * T
TB = 1024
G = N // TB


def _gate_body(x_ref, wg_ref, bg_ref, eid_ref, psum_ref, cnt_ref):
    xb = x_ref[...]
    logits = jnp.dot(xb, wg_ref[...], preferred_element_type=jnp.float32)
    logits = logits + bg_ref[...]
    m = jnp.max(logits, axis=-1, keepdims=True)
    ex = jnp.exp(logits - m)
    s = jnp.sum(ex, axis=-1, keepdims=True)
    probs = ex / s
    ids = lax.broadcasted_iota(jnp.int32, logits.shape, 1)
    eid = jnp.min(jnp.where(logits == m, ids, E), axis=-1, keepdims=True)
    eid_ref[...] = eid
    psum_ref[...] = jnp.sum(probs, axis=0, keepdims=True)[None]
    onehot = (ids == eid).astype(jnp.float32)
    cnt_ref[...] = jnp.sum(onehot, axis=0, keepdims=True)[None]


def _gate_call(x, Wg, bg2):
    return pl.pallas_call(
        _gate_body,
        grid=(G,),
        in_specs=[
            pl.BlockSpec((TB, H), lambda i: (i, 0)),
            pl.BlockSpec((H, E), lambda i: (0, 0)),
            pl.BlockSpec((1, E), lambda i: (0, 0)),
        ],
        out_specs=[
            pl.BlockSpec((TB, 1), lambda i: (i, 0)),
            pl.BlockSpec((1, 1, E), lambda i: (i, 0, 0)),
            pl.BlockSpec((1, 1, E), lambda i: (i, 0, 0)),
        ],
        out_shape=[
            jax.ShapeDtypeStruct((N, 1), jnp.int32),
            jax.ShapeDtypeStruct((G, 1, E), jnp.float32),
            jax.ShapeDtypeStruct((G, 1, E), jnp.float32),
        ],
    )(x, Wg, bg2)


def _route_metadata(eid, counts):
    perm = jnp.argsort(eid).astype(jnp.int32)
    ntiles = (counts + (T - 1)) // T
    cum_tiles = jnp.cumsum(ntiles)
    padded_start = (cum_tiles - ntiles) * T
    orig_start = jnp.cumsum(counts) - counts
    r = jnp.arange(N, dtype=jnp.int32)
    e_sorted = eid[perm]
    slot = padded_start[e_sorted] + (r - orig_start[e_sorted])
    pad_idx = jnp.arange(NP, dtype=jnp.int32) & (N - 1)
    src_idx = pad_idx.at[slot].set(perm)
    pos = jnp.zeros((N,), jnp.int32).at[perm].set(slot)
    tile_ids = jnp.arange(NT, dtype=jnp.int32)
    expert_tile = jnp.minimum(
        jnp.searchsorted(cum_tiles, tile_ids, side="right"), E - 1
    ).astype(jnp.int32)
    return src_idx, pos, expert_tile


def _expert_body(et_ref, xs_ref, w1_ref, b1_ref, w2_ref, b2_ref,
                 wg_ref, bg_ref, ys_ref):
    del et_ref
    xb = xs_ref[...]
    logits = jnp.dot(xb, wg_ref[...], preferred_element_type=jnp.float32)
    logits = logits + bg_ref[...]
    m = jnp.max(logits, axis=-1, keepdims=True)
    p = 1.0 / jnp.sum(jnp.exp(logits - m), axis=-1, keepdims=True)
    h = jnp.dot(xb, w1_ref[0], preferred_element_type=jnp.float32)
    h = jnp.maximum(h + b1_ref[0], 0.0)
    o = jnp.dot(h, w2_ref[0], preferred_element_type=jnp.float32)
    ys_ref[...] = (o + b2_ref[0]) * p


def _expert_call(expert_tile, xs, W1, b1, W2, b2, Wg, bg2):
    grid_spec = pltpu.PrefetchScalarGridSpec(
        num_scalar_prefetch=1,
        grid=(NT,),
        in_specs=[
            pl.BlockSpec((T, H), lambda i, et: (i, 0)),
            pl.BlockSpec((1, H, F), lambda i, et: (et[i], 0, 0)),
            pl.BlockSpec((1, 1, F), lambda i, et: (et[i], 0, 0)),
            pl.BlockSpec((1, F, H), lambda i, et: (et[i], 0, 0)),
            pl.BlockSpec((1, 1, H), lambda i, et: (et[i], 0, 0)),
            pl.BlockSpec((H, E), lambda i, et: (0, 0)),
            pl.BlockSpec((1, E), lambda i, et: (0, 0)),
        ],
        out_specs=pl.BlockSpec((T, H), lambda i, et: (i, 0)),
    )
    return pl.pallas_call(
        _expert_body,
        grid_spec=grid_spec,
        out_shape=jax.ShapeDtypeStruct((NP, H), jnp.float32),
        compiler_params=pltpu.CompilerParams(
            dimension_semantics=("arbitrary",),
        ),
    )(expert_tile, xs, W1, b1, W2, b2, Wg, bg2)


@functools.lru_cache(maxsize=None)
def _make_sc_gather(n_rows_out, d):
    info = plsc.get_sparse_core_info()
    _NC = info.num_cores
    _NW = info.num_cores * info.num_subcores
    b_per_w = n_rows_out // _NW
    ch = 64 if b_per_w % 64 == 0 else b_per_w
    n_ch = b_per_w // ch
    mesh = plsc.VectorSubcoreMesh(core_axis_name="c", subcore_axis_name="s")

    @functools.partial(
        pl.kernel,
        mesh=mesh,
        out_type=jax.ShapeDtypeStruct((n_rows_out, d), jnp.float32),
        scratch_types=[
            pltpu.VMEM((b_per_w,), jnp.int32),
            pltpu.VMEM((ch, d), jnp.float32),
            pltpu.SemaphoreType.DMA,
        ],
    )
    def k(table_hbm, idx_hbm, out_hbm, idx_v, rows_v, sem):
        wid = lax.axis_index("s") * _NC + lax.axis_index("c")
        base = wid * b_per_w
        pltpu.sync_copy(idx_hbm.at[pl.ds(base, b_per_w)], idx_v)
        for c in range(n_ch):
            pltpu.async_copy(
                table_hbm.at[idx_v.at[pl.ds(c * ch, ch)]], rows_v, sem
            ).wait()
            pltpu.sync_copy(rows_v, out_hbm.at[pl.ds(base + c * ch, ch)])

    return k


def _gather_rows(table, idx):
    return _make_sc_gather(idx.shape[0], table.shape[1])(table, idx)


def kernel(x, Wg, bg, W1, b1, W2, b2):
    bg2 = bg.reshape(1, E)
    eid2, psum, cnt = _gate_call(x, Wg, bg2)
    eid = eid2[:, 0]
    probs_sum = psum.sum(axis=(0, 1))
    counts_f = cnt.sum(axis=(0, 1))
    aux = jnp.dot(probs_sum / N, counts_f / N) * E
    counts = counts_f.astype(jnp.int32)
    src_idx, pos, expert_tile = _route_metadata(eid, counts)
    xs = _gather_rows(x, src_idx)
    ys = _expert_call(expert_tile, xs, W1, b1[:, None, :], W2,
                      b2[:, None, :], Wg, bg2)
    y = _gather_rows(ys, pos)
    return y, aux
